# baseline (speedup 1.0000x reference)
import functools

import jax
import jax.numpy as jnp
from jax import lax
from jax.experimental import pallas as pl
from jax.experimental.pallas import tpu as pltpu
from jax.experimental.pallas import tpu_sc as plsc

DIM = 768
EPS = 1e-9
L = 16
KV = DIM // L
NC, NS = 2, 16
NW = NC * NS


def _rsqrt(v):
    i = lax.bitcast_convert_type(v, jnp.int32)
    i = jnp.full((L,), 0x5F3759DF, jnp.int32) - lax.shift_right_logical(i, 1)
    y = lax.bitcast_convert_type(i, jnp.float32)
    half = v * 0.5
    for _ in range(3):
        y = y * (1.5 - half * y * y)
    return y


def _lane_allreduce_sum(s, scratch):
    lanes = lax.iota(jnp.int32, L)
    for m in (1, 2, 4, 8):
        scratch[...] = s
        s = s + plsc.load_gather(scratch, [lanes ^ m])
    return s


def _make_sc_kernel(T, N, C):
    RW = T // NW
    NCHUNK = RW // C
    mesh = plsc.VectorSubcoreMesh(core_axis_name="c", subcore_axis_name="s")

    @functools.partial(
        pl.kernel,
        out_type=(
            jax.ShapeDtypeStruct((T, DIM), jnp.float32),
            jax.ShapeDtypeStruct((T, DIM), jnp.float32),
        ),
        mesh=mesh,
        compiler_params=pltpu.CompilerParams(needs_layout_passes=False),
        scratch_types=[
            pltpu.VMEM((RW,), jnp.int32),
            pltpu.VMEM((C, DIM), jnp.float32),
            pltpu.VMEM((C, DIM), jnp.float32),
            pltpu.VMEM((DIM,), jnp.float32),
            pltpu.VMEM((DIM,), jnp.float32),
            pltpu.VMEM((L,), jnp.float32),
            pltpu.SemaphoreType.DMA,
        ],
    )
    def body(ids_hbm, wt_hbm, pt_hbm, g_hbm, b_hbm, out1_hbm, out2_hbm,
             idx_v, wbuf, pbuf, gbuf, bbuf, red, sem):
        wid = lax.axis_index("s") * NC + lax.axis_index("c")
        base = wid * RW
        pos_base = base % N
        pltpu.sync_copy(ids_hbm.at[pl.ds(base, RW)], idx_v)
        pltpu.sync_copy(g_hbm, gbuf)
        pltpu.sync_copy(b_hbm, bbuf)

        def chunk_body(j, _):
            row0 = base + j * C
            prow0 = pos_base + j * C
            pltpu.async_copy(wt_hbm.at[idx_v.at[pl.ds(j * C, C)]], wbuf,
                             sem).wait()
            pltpu.sync_copy(pt_hbm.at[pl.ds(prow0, C)], pbuf)

            def row_body(r, _):
                def p1(k, carry):
                    s, ss = carry
                    x = wbuf[r, pl.ds(k * L, L)] + pbuf[r, pl.ds(k * L, L)]
                    wbuf[r, pl.ds(k * L, L)] = x
                    return (s + x, ss + x * x)

                zero = jnp.zeros((L,), jnp.float32)
                s, ss = lax.fori_loop(0, KV, p1, (zero, zero))
                mean = _lane_allreduce_sum(s, red) * (1.0 / DIM)
                ex2 = _lane_allreduce_sum(ss, red) * (1.0 / DIM)
                rstd = _rsqrt(ex2 - mean * mean + EPS)

                def p2(k, c):
                    x = wbuf[r, pl.ds(k * L, L)]
                    g = gbuf[pl.ds(k * L, L)]
                    bb = bbuf[pl.ds(k * L, L)]
                    wbuf[r, pl.ds(k * L, L)] = (x - mean) * rstd * g + bb
                    return c

                lax.fori_loop(0, KV, p2, 0)
                return 0

            lax.fori_loop(0, C, row_body, 0)
            pltpu.sync_copy(wbuf, out1_hbm.at[pl.ds(row0, C)])
            pltpu.sync_copy(pbuf, out2_hbm.at[pl.ds(row0, C)])
            return 0

        lax.fori_loop(0, NCHUNK, chunk_body, 0)

    return body


@jax.jit
def kernel(input_ids, word_table, pos_table, ln_gamma, ln_beta):
    b, n = input_ids.shape
    T = b * n
    ids = input_ids.reshape(T).astype(jnp.int32)
    sc = _make_sc_kernel(T, n, 64)
    out1, out2 = sc(ids, word_table, pos_table, ln_gamma, ln_beta)
    return out1.reshape(b, n, DIM), out2.reshape(b, n, DIM)


# speedup vs baseline: 1.8937x; 1.8937x over previous
import functools

import jax
import jax.numpy as jnp
from jax import lax
from jax.experimental import pallas as pl
from jax.experimental.pallas import tpu as pltpu
from jax.experimental.pallas import tpu_sc as plsc

DIM = 768
EPS = 1e-9
L = 16
KV = DIM // L
NC, NS = 2, 16
NW = NC * NS
NSLOT = 3


def _rsqrt(v):
    i = lax.bitcast_convert_type(v, jnp.int32)
    i = jnp.full((L,), 0x5F3759DF, jnp.int32) - lax.shift_right_logical(i, 1)
    y = lax.bitcast_convert_type(i, jnp.float32)
    half = v * 0.5
    for _ in range(3):
        y = y * (1.5 - half * y * y)
    return y


def _lane_allreduce2(s, ss, scr, scr2):
    lanes = lax.iota(jnp.int32, L)
    for m in (1, 2, 4, 8):
        scr[...] = s
        scr2[...] = ss
        s = s + plsc.load_gather(scr, [lanes ^ m])
        ss = ss + plsc.load_gather(scr2, [lanes ^ m])
    return s, ss


def _make_sc_kernel(T, N, C):
    RW = T // NW
    NCHUNK = RW // C
    mesh = plsc.VectorSubcoreMesh(core_axis_name="c", subcore_axis_name="s")

    @functools.partial(
        pl.kernel,
        out_type=(
            jax.ShapeDtypeStruct((T, DIM), jnp.float32),
            jax.ShapeDtypeStruct((T, DIM), jnp.float32),
        ),
        mesh=mesh,
        compiler_params=pltpu.CompilerParams(needs_layout_passes=False),
        scratch_types=[
            pltpu.VMEM((RW,), jnp.int32),
            pltpu.VMEM((NSLOT, C, DIM), jnp.float32),
            pltpu.VMEM((NSLOT, C, DIM), jnp.float32),
            pltpu.VMEM((DIM,), jnp.float32),
            pltpu.VMEM((DIM,), jnp.float32),
            pltpu.VMEM((L,), jnp.float32),
            pltpu.VMEM((L,), jnp.float32),
            pltpu.SemaphoreType.DMA((NSLOT,)),
            pltpu.SemaphoreType.DMA((NSLOT,)),
            pltpu.SemaphoreType.DMA((NSLOT,)),
        ],
    )
    def body(ids_hbm, wt_hbm, pt_hbm, g_hbm, b_hbm, out1_hbm, out2_hbm,
             idx_v, wbuf, pbuf, gbuf, bbuf, red, red2, sw, sp, so):
        wid = lax.axis_index("s") * NC + lax.axis_index("c")
        base = wid * RW
        pos_base = base % N
        pltpu.sync_copy(ids_hbm.at[pl.ds(base, RW)], idx_v)
        pltpu.sync_copy(g_hbm, gbuf)
        pltpu.sync_copy(b_hbm, bbuf)

        def start_in(j, slot):
            pltpu.async_copy(wt_hbm.at[idx_v.at[pl.ds(j * C, C)]],
                             wbuf.at[slot], sw.at[slot])
            pltpu.async_copy(pt_hbm.at[pl.ds(pos_base + j * C, C)],
                             pbuf.at[slot], sp.at[slot])

        def wait_in(j, slot):
            pltpu.make_async_copy(wt_hbm.at[idx_v.at[pl.ds(j * C, C)]],
                                  wbuf.at[slot], sw.at[slot]).wait()
            pltpu.make_async_copy(pt_hbm.at[pl.ds(pos_base + j * C, C)],
                                  pbuf.at[slot], sp.at[slot]).wait()

        def start_out(j, slot):
            row0 = base + j * C
            pltpu.async_copy(wbuf.at[slot], out1_hbm.at[pl.ds(row0, C)],
                             so.at[slot])
            pltpu.async_copy(pbuf.at[slot], out2_hbm.at[pl.ds(row0, C)],
                             so.at[slot])

        def wait_out(j, slot):
            row0 = base + j * C
            pltpu.make_async_copy(wbuf.at[slot], out1_hbm.at[pl.ds(row0, C)],
                                  so.at[slot]).wait()
            pltpu.make_async_copy(pbuf.at[slot], out2_hbm.at[pl.ds(row0, C)],
                                  so.at[slot]).wait()

        def compute(slot):
            def row_body(r, carry):
                acc = [jnp.zeros((L,), jnp.float32) for _ in range(8)]
                for k in range(KV):
                    w = wbuf[slot, r, pl.ds(k * L, L)]
                    p = pbuf[slot, r, pl.ds(k * L, L)]
                    x = w + p
                    wbuf[slot, r, pl.ds(k * L, L)] = x
                    acc[k % 4] = acc[k % 4] + x
                    acc[4 + k % 4] = acc[4 + k % 4] + x * x
                s = (acc[0] + acc[1]) + (acc[2] + acc[3])
                ss = (acc[4] + acc[5]) + (acc[6] + acc[7])
                s, ss = _lane_allreduce2(s, ss, red, red2)
                mean = s * (1.0 / DIM)
                ex2 = ss * (1.0 / DIM)
                rstd = _rsqrt(ex2 - mean * mean + EPS)
                for k in range(KV):
                    x = wbuf[slot, r, pl.ds(k * L, L)]
                    g = gbuf[pl.ds(k * L, L)]
                    bb = bbuf[pl.ds(k * L, L)]
                    wbuf[slot, r, pl.ds(k * L, L)] = \
                        (x - mean) * rstd * g + bb
                return carry
            lax.fori_loop(0, C, row_body, 0)

        start_in(0, 0)

        def chunk(j, carry):
            slot = lax.rem(j, NSLOT)
            nslot = lax.rem(j + 1, NSLOT)

            @pl.when(j >= 2)
            def _():
                wait_out(j - 2, nslot)

            @pl.when(j + 1 < NCHUNK)
            def _():
                start_in(j + 1, nslot)

            wait_in(j, slot)
            compute(slot)
            start_out(j, slot)
            return carry

        lax.fori_loop(0, NCHUNK, chunk, 0)
        wait_out(NCHUNK - 2, (NCHUNK - 2) % NSLOT)
        wait_out(NCHUNK - 1, (NCHUNK - 1) % NSLOT)

    return body


@jax.jit
def kernel(input_ids, word_table, pos_table, ln_gamma, ln_beta):
    b, n = input_ids.shape
    T = b * n
    ids = input_ids.reshape(T).astype(jnp.int32)
    sc = _make_sc_kernel(T, n, 16)
    out1, out2 = sc(ids, word_table, pos_table, ln_gamma, ln_beta)
    return out1.reshape(b, n, DIM), out2.reshape(b, n, DIM)
